# baseline (speedup 1.0000x reference)
import functools

import jax
import jax.numpy as jnp
from jax import lax
from jax.experimental import pallas as pl
from jax.experimental.pallas import tpu as pltpu
from jax.experimental.pallas import tpu_sc as plsc

B = 16384
D = 64
V = 1000000
NC = 2
NS = 16
NW = NC * NS
RSHIFT = 15
RANGE = 1 << RSHIFT
GRP = 512
NGRP = RANGE // GRP
TILE = 128
SCAN_CH = 2048
NCHUNK = B // SCAN_CH
OCAP = 8192
LCAP = 2048
SCH = 512
SENT = 0x7FFF0000
LAST_TILE_BASE = (V // TILE) * TILE


def _sc_embed(c, table_t):
    mesh = plsc.VectorSubcoreMesh(core_axis_name="c", subcore_axis_name="s")

    @functools.partial(
        pl.kernel,
        mesh=mesh,
        out_type=jax.ShapeDtypeStruct((B * D + D,), jnp.float32),
        compiler_params=pltpu.CompilerParams(use_tc_tiling_on_sc=True, needs_layout_passes=False),
        scratch_types=[
            pltpu.VMEM((SCAN_CH,), jnp.int32),
            pltpu.VMEM((OCAP + 32,), jnp.int32),
            pltpu.VMEM((OCAP + 32,), jnp.int32),
            pltpu.VMEM((2, D, GRP), jnp.float32),
            pltpu.VMEM((LCAP + 32,), jnp.int32),
            pltpu.VMEM((LCAP + 32,), jnp.int32),
            pltpu.VMEM((2, 8, 128), jnp.int32),
            pltpu.VMEM((2, 8, 128), jnp.float32),
            pltpu.SemaphoreType.DMA,
            pltpu.SemaphoreType.DMA,
        ],
    )
    def k(c_hbm, tbl_hbm, out_hbm, cch_v, oj_v, oc_v, slab_v, sj_v, sc_v,
          idxb_v, valb_v, sem_slab, sem_out):
        wid = lax.axis_index("s") * NC + lax.axis_index("c")
        rbase = wid * RANGE
        span = jnp.maximum(jnp.minimum(V - rbase, RANGE), 0)
        ngrp = (span + GRP - 1) // GRP
        lanes = lax.iota(jnp.int32, 16)

        def grp_base(g):
            return pl.multiple_of(rbase + g * GRP, GRP)

        def sub_ok(g, d):
            return grp_base(g) + d * TILE <= LAST_TILE_BASE

        def slab_dma(g, d):
            par = g & 1
            return pltpu.make_async_copy(
                tbl_hbm.at[:, pl.ds(pl.multiple_of(grp_base(g) + d * TILE, TILE), TILE)],
                slab_v.at[par, :, pl.ds(d * TILE, TILE)],
                sem_slab,
            )

        def start_group(g):
            for d in range(4):
                @pl.when(sub_ok(g, d))
                def _():
                    slab_dma(g, d).start()

        def wait_group(g):
            for d in range(4):
                @pl.when(sub_ok(g, d))
                def _():
                    slab_dma(g, d).wait()

        def scat_wait():
            pltpu.make_async_copy(
                valb_v.at[0, 0], out_hbm.at[idxb_v.at[0, 0]], sem_out
            ).wait()

        def outer_body(gp, full_total_prev):
            lo = gp * OCAP

            def chunk_body(ci, carry):
                off, seen = carry
                pltpu.sync_copy(c_hbm.at[pl.ds(ci * SCAN_CH, SCAN_CH)], cch_v)

                def vec_body(kk, carry2):
                    off2, seen2 = carry2
                    c16 = cch_v[pl.ds(kk * 16, 16)]
                    m = (c16 >> RSHIFT) == wid
                    mi = m.astype(jnp.int32)
                    incl = plsc.cumsum(mi)
                    rank = seen2 + incl - mi
                    mm = m & (rank >= lo) & (rank < lo + OCAP)
                    n = jnp.sum(mm.astype(jnp.int32))
                    plsc.store_compressed(oc_v.at[pl.ds(off2, 16)], c16, mask=mm)
                    jv = lanes + (ci * SCAN_CH + kk * 16)
                    plsc.store_compressed(oj_v.at[pl.ds(off2, 16)], jv, mask=mm)
                    return off2 + n, seen2 + jnp.sum(mi)

                return lax.fori_loop(0, SCAN_CH // 16, vec_body, (off, seen))

            total, full_total = lax.fori_loop(0, NCHUNK, chunk_body,
                                              (jnp.int32(0), jnp.int32(0)))
            oc_v[pl.ds(total, 16)] = jnp.full((16,), SENT, jnp.int32)
            nblk = (total + 15) // 16

            @pl.when(ngrp > 0)
            def _():
                start_group(0)

            def group_body(g, gcarry):
                wait_group(g)

                @pl.when(g + 1 < ngrp)
                def _():
                    start_group(g + 1)

                par = g & 1
                base = rbase + g * GRP
                gid = wid * NGRP + g

                def inner_cond(st):
                    p, again = st
                    return again

                def inner_body(st):
                    p, _ = st
                    llo = p * LCAP

                    def blk_body(bb, carry2):
                        cnt2, seen2 = carry2
                        oc16 = oc_v[pl.ds(bb * 16, 16)]
                        m = (oc16 >> 9) == gid
                        mi = m.astype(jnp.int32)
                        incl = plsc.cumsum(mi)
                        rank = seen2 + incl - mi
                        mm = m & (rank >= llo) & (rank < llo + LCAP)
                        n = jnp.sum(mm.astype(jnp.int32))
                        oj16 = oj_v[pl.ds(bb * 16, 16)]
                        plsc.store_compressed(sc_v.at[pl.ds(cnt2, 16)], oc16, mask=mm)
                        plsc.store_compressed(sj_v.at[pl.ds(cnt2, 16)], oj16, mask=mm)
                        return cnt2 + n, seen2 + jnp.sum(mi)

                    cnt, _seen = lax.fori_loop(0, nblk, blk_body,
                                               (jnp.int32(0), jnp.int32(0)))

                    sj_v[pl.ds(cnt, 16)] = jnp.full((16,), B, jnp.int32)
                    sc_v[pl.ds(cnt, 16)] = jnp.full((16,), base, jnp.int32)

                    nblk16 = (cnt + 15) // 16

                    def eblock(b, ecarry):
                        par2 = b & 1

                        @pl.when(b >= 2)
                        def _():
                            for _q in range(8):
                                scat_wait()

                        j16 = sj_v[pl.ds(b * 16, 16)]
                        c16 = sc_v[pl.ds(b * 16, 16)]
                        col16 = c16 - base
                        jD = j16 * D
                        for d in range(D):
                            vals = plsc.load_gather(
                                slab_v.at[par],
                                [jnp.full((16,), d, jnp.int32), col16])
                            q, r = d // 8, (d % 8) * 16
                            valb_v[par2, q, pl.ds(r, 16)] = vals
                            idxb_v[par2, q, pl.ds(r, 16)] = jD + d
                        for q in range(8):
                            pltpu.async_copy(
                                valb_v.at[par2, q],
                                out_hbm.at[idxb_v.at[par2, q]],
                                sem_out)
                        return ecarry

                    lax.fori_loop(0, nblk16, eblock, 0)
                    ndrain = jnp.minimum(nblk16, 2) * 8

                    def drain_body(_i, _c2):
                        scat_wait()
                        return _c2

                    lax.fori_loop(0, ndrain, drain_body, 0)
                    return p + 1, cnt >= LCAP

                lax.while_loop(inner_cond, inner_body,
                               (jnp.int32(0), jnp.bool_(True)))
                return gcarry

            lax.fori_loop(0, ngrp, group_body, 0)
            return full_total

        full_total = outer_body(jnp.int32(0), jnp.int32(0))
        npass = (full_total + OCAP - 1) // OCAP

        def extra_pass(gp, carry):
            return outer_body(gp, carry)

        lax.fori_loop(1, npass, extra_pass, full_total)

    return k(c, table_t)


def kernel(c, table):
    out_flat = _sc_embed(c, table.T)
    return out_flat[: B * D].reshape(B, D)


# speedup vs baseline: 530.9767x; 530.9767x over previous
import functools

import jax
import jax.numpy as jnp
from jax import lax
from jax.experimental import pallas as pl
from jax.experimental.pallas import tpu as pltpu
from jax.experimental.pallas import tpu_sc as plsc

B = 16384
D = 64
V = 1000000
NC = 2
NS = 16
NW = NC * NS
RSHIFT = 15
RANGE = 1 << RSHIFT
GRP = 512
NGRP = RANGE // GRP
TILE = 128
SCAN_CH = 2048
NCHUNK = B // SCAN_CH
OCAP = 8192
LCAP = 2048
CAP = 16640
SENT = 0x7FFF0000
LAST_TILE_BASE = (V // TILE) * TILE


def _sc_collect(c, table_t):
    mesh = plsc.VectorSubcoreMesh(core_axis_name="c", subcore_axis_name="s")

    @functools.partial(
        pl.kernel,
        mesh=mesh,
        out_type=(
            jax.ShapeDtypeStruct((NW * CAP * D,), jnp.float32),
            jax.ShapeDtypeStruct((NW * CAP,), jnp.int32),
        ),
        compiler_params=pltpu.CompilerParams(
            use_tc_tiling_on_sc=True, needs_layout_passes=False),
        scratch_types=[
            pltpu.VMEM((SCAN_CH,), jnp.int32),
            pltpu.VMEM((OCAP + 32,), jnp.int32),
            pltpu.VMEM((OCAP + 32,), jnp.int32),
            pltpu.VMEM((2, D, GRP), jnp.float32),
            pltpu.VMEM((LCAP + 32,), jnp.int32),
            pltpu.VMEM((LCAP + 32,), jnp.int32),
            pltpu.VMEM((2 * 16 * D,), jnp.float32),
            pltpu.VMEM((CAP,), jnp.int32),
            pltpu.SemaphoreType.DMA,
            pltpu.SemaphoreType.DMA,
        ],
    )
    def k1(c_hbm, tbl_hbm, vals_hbm, jl_hbm, cch_v, oj_v, oc_v, slab_v,
           sj_v, sc_v, row_v, jlf_v, sem_slab, sem_out):
        wid = lax.axis_index("s") * NC + lax.axis_index("c")
        rbase = wid * RANGE
        span = jnp.maximum(jnp.minimum(V - rbase, RANGE), 0)
        ngrp = (span + GRP - 1) // GRP
        lanes = lax.iota(jnp.int32, 16)
        vbase = wid * CAP

        def grp_base(g):
            return pl.multiple_of(rbase + g * GRP, GRP)

        def sub_ok(g, d):
            return grp_base(g) + d * TILE <= LAST_TILE_BASE

        def slab_dma(g, d):
            return pltpu.make_async_copy(
                tbl_hbm.at[:, pl.ds(
                    pl.multiple_of(grp_base(g) + d * TILE, TILE), TILE)],
                slab_v.at[g & 1, :, pl.ds(d * TILE, TILE)],
                sem_slab,
            )

        def start_group(g):
            for d in range(4):
                @pl.when(sub_ok(g, d))
                def _():
                    slab_dma(g, d).start()

        def wait_group(g):
            for d in range(4):
                @pl.when(sub_ok(g, d))
                def _():
                    slab_dma(g, d).wait()

        def row_wait():
            pltpu.make_async_copy(
                row_v.at[pl.ds(0, 16 * D)],
                vals_hbm.at[pl.ds(0, 16 * D)], sem_out).wait()

        def outer_body(gp, done_in):
            lo = gp * OCAP

            def chunk_body(ci, carry):
                off, seen = carry
                pltpu.sync_copy(c_hbm.at[pl.ds(ci * SCAN_CH, SCAN_CH)], cch_v)

                def vec_body(kk, carry2):
                    off2, seen2 = carry2
                    c16 = cch_v[pl.ds(kk * 16, 16)]
                    m = (c16 >> RSHIFT) == wid
                    mi = m.astype(jnp.int32)
                    incl = plsc.cumsum(mi)
                    rank = seen2 + incl - mi
                    mm = m & (rank >= lo) & (rank < lo + OCAP)
                    n = jnp.sum(mm.astype(jnp.int32))
                    plsc.store_compressed(oc_v.at[pl.ds(off2, 16)], c16,
                                          mask=mm)
                    jv = lanes + (ci * SCAN_CH + kk * 16)
                    plsc.store_compressed(oj_v.at[pl.ds(off2, 16)], jv,
                                          mask=mm)
                    return off2 + n, seen2 + jnp.sum(mi)

                return lax.fori_loop(0, SCAN_CH // 16, vec_body, (off, seen))

            total, full_total = lax.fori_loop(
                0, NCHUNK, chunk_body, (jnp.int32(0), jnp.int32(0)))
            oc_v[pl.ds(total, 16)] = jnp.full((16,), SENT, jnp.int32)
            nblk = (total + 15) // 16

            @pl.when(ngrp > 0)
            def _():
                start_group(0)

            def group_body(g, done_g):
                wait_group(g)

                @pl.when(g + 1 < ngrp)
                def _():
                    start_group(g + 1)

                par = g & 1
                base = rbase + g * GRP
                gid = wid * NGRP + g

                def inner_cond(st):
                    return st[1]

                def inner_body(st):
                    p, _a, done_p = st
                    llo = p * LCAP

                    def blk_body(bb, carry2):
                        cnt2, seen2 = carry2
                        oc16 = oc_v[pl.ds(bb * 16, 16)]
                        m = (oc16 >> 9) == gid
                        mi = m.astype(jnp.int32)
                        incl = plsc.cumsum(mi)
                        rank = seen2 + incl - mi
                        mm = m & (rank >= llo) & (rank < llo + LCAP)
                        n = jnp.sum(mm.astype(jnp.int32))
                        oj16 = oj_v[pl.ds(bb * 16, 16)]
                        plsc.store_compressed(sc_v.at[pl.ds(cnt2, 16)],
                                              oc16, mask=mm)
                        plsc.store_compressed(sj_v.at[pl.ds(cnt2, 16)],
                                              oj16, mask=mm)
                        return cnt2 + n, seen2 + jnp.sum(mi)

                    cnt, _seen = lax.fori_loop(0, nblk, blk_body,
                                               (jnp.int32(0), jnp.int32(0)))
                    sj_v[pl.ds(cnt, 16)] = jnp.full((16,), B, jnp.int32)
                    sc_v[pl.ds(cnt, 16)] = jnp.full((16,), base, jnp.int32)
                    nblk16 = (cnt + 15) // 16

                    def eblock(b, ecarry):
                        par2 = b & 1

                        @pl.when(b >= 2)
                        def _():
                            row_wait()

                        c16 = sc_v[pl.ds(b * 16, 16)]
                        col16 = c16 - base
                        for d in range(D):
                            vals = plsc.load_gather(
                                slab_v.at[par],
                                [jnp.full((16,), d, jnp.int32), col16])
                            plsc.store_scatter(
                                row_v,
                                [par2 * (16 * D) + lanes * D + d], vals)
                        dst0 = (vbase + done_p + b * 16) * D
                        pltpu.async_copy(
                            row_v.at[pl.ds(par2 * (16 * D), 16 * D)],
                            vals_hbm.at[pl.ds(dst0, 16 * D)],
                            sem_out)
                        jlf_v[pl.ds(done_p + b * 16, 16)] = sj_v[
                            pl.ds(b * 16, 16)]
                        return ecarry

                    lax.fori_loop(0, nblk16, eblock, 0)
                    ndrain = jnp.minimum(nblk16, 2)

                    def drain_body(_i, _c2):
                        row_wait()
                        return _c2

                    lax.fori_loop(0, ndrain, drain_body, 0)
                    return p + 1, cnt >= LCAP, done_p + cnt

                _p, _a, done_g2 = lax.while_loop(
                    inner_cond, inner_body,
                    (jnp.int32(0), jnp.bool_(True), done_g))
                return done_g2

            done_out = lax.fori_loop(0, ngrp, group_body, done_in)
            return full_total, done_out

        full_total, done = outer_body(jnp.int32(0), jnp.int32(0))
        npass = (full_total + OCAP - 1) // OCAP

        def extra_pass(gp, carry):
            _ft, done2 = outer_body(gp, carry)
            return done2

        done = lax.fori_loop(1, npass, extra_pass, done)
        for t in range(10):
            jlf_v[pl.ds(done + t * 16, 16)] = jnp.full((16,), B, jnp.int32)
        pltpu.sync_copy(jlf_v, jl_hbm.at[pl.ds(vbase, CAP)])

    return k1(c, table_t)


def _sc_scatter(c, vals2, jl):
    mesh = plsc.VectorSubcoreMesh(core_axis_name="c", subcore_axis_name="s")

    @functools.partial(
        pl.kernel,
        mesh=mesh,
        out_type=jax.ShapeDtypeStruct((B + 16, D), jnp.float32),
        compiler_params=pltpu.CompilerParams(
            use_tc_tiling_on_sc=False, needs_layout_passes=False),
        scratch_types=[
            pltpu.VMEM((SCAN_CH,), jnp.int32),
            pltpu.VMEM((2, TILE), jnp.int32),
            pltpu.VMEM((2, TILE, D), jnp.float32),
            pltpu.SemaphoreType.DMA,
            pltpu.SemaphoreType.DMA,
        ],
    )
    def k2(c_hbm, vals_hbm, jl_hbm, out_hbm, cch_v, jl_v, row_v,
           sem_in, sem_sc):
        wid = lax.axis_index("s") * NC + lax.axis_index("c")
        vbase = wid * CAP

        def chunk_body(ci, cnt):
            pltpu.sync_copy(c_hbm.at[pl.ds(ci * SCAN_CH, SCAN_CH)], cch_v)

            def vec_body(kk, cnt2):
                c16 = cch_v[pl.ds(kk * 16, 16)]
                m = (c16 >> RSHIFT) == wid
                return cnt2 + jnp.sum(m.astype(jnp.int32))

            return lax.fori_loop(0, SCAN_CH // 16, vec_body, cnt)

        count = lax.fori_loop(0, NCHUNK, chunk_body, jnp.int32(0))
        nch = (count + TILE - 1) // TILE

        def in_dma(ch):
            par = ch & 1
            a = pltpu.make_async_copy(
                jl_hbm.at[pl.ds(vbase + ch * TILE, TILE)],
                jl_v.at[par], sem_in)
            b = pltpu.make_async_copy(
                vals_hbm.at[pl.ds(vbase + ch * TILE, TILE), :],
                row_v.at[par], sem_in)
            return a, b

        def start_in(ch):
            a, b = in_dma(ch)
            a.start()
            b.start()

        def wait_in(ch):
            a, b = in_dma(ch)
            a.wait()
            b.wait()

        def scat_wait():
            pltpu.make_async_copy(
                row_v.at[0], out_hbm.at[jl_v.at[0]], sem_sc).wait()

        @pl.when(nch > 0)
        def _():
            start_in(0)

        def ch_body(ch, carry):
            par = ch & 1
            wait_in(ch)

            @pl.when(ch + 1 < nch)
            def _():
                start_in(ch + 1)

            @pl.when(ch >= 2)
            def _():
                scat_wait()

            pltpu.async_copy(
                row_v.at[par], out_hbm.at[jl_v.at[par]], sem_sc)
            return carry

        lax.fori_loop(0, nch, ch_body, 0)
        ndrain = jnp.minimum(nch, 2)

        def drain_body(_i, _c):
            scat_wait()
            return _c

        lax.fori_loop(0, ndrain, drain_body, 0)

    return k2(c, vals2, jl)


def kernel(c, table):
    vals_flat, jl = _sc_collect(c, table.T)
    out = _sc_scatter(c, vals_flat.reshape(NW * CAP, D), jl)
    return out[:B]


# speedup vs baseline: 542.7188x; 1.0221x over previous
import functools

import jax
import jax.numpy as jnp
from jax import lax
from jax.experimental import pallas as pl
from jax.experimental.pallas import tpu as pltpu
from jax.experimental.pallas import tpu_sc as plsc

B = 16384
D = 64
V = 1000000
NC = 2
NS = 16
NW = NC * NS
RSHIFT = 15
RANGE = 1 << RSHIFT
GRP = 512
NGRP = RANGE // GRP
TILE = 128
SCAN_CH = 2048
NCHUNK = B // SCAN_CH
OCAP = 8192
LCAP = 2048
CAP = 16640
SENT = 0x7FFF0000
LAST_TILE_BASE = (V // TILE) * TILE


def _sc_collect(c, table_t):
    mesh = plsc.VectorSubcoreMesh(core_axis_name="c", subcore_axis_name="s")

    @functools.partial(
        pl.kernel,
        mesh=mesh,
        out_type=(
            jax.ShapeDtypeStruct((NW * CAP * D,), jnp.float32),
            jax.ShapeDtypeStruct((NW * CAP,), jnp.int32),
        ),
        compiler_params=pltpu.CompilerParams(
            use_tc_tiling_on_sc=True, needs_layout_passes=False),
        scratch_types=[
            pltpu.VMEM((SCAN_CH,), jnp.int32),
            pltpu.VMEM((OCAP + 96,), jnp.int32),
            pltpu.VMEM((OCAP + 96,), jnp.int32),
            pltpu.VMEM((2, D, GRP), jnp.float32),
            pltpu.VMEM((LCAP + 32,), jnp.int32),
            pltpu.VMEM((LCAP + 32,), jnp.int32),
            pltpu.VMEM((2 * 16 * D,), jnp.float32),
            pltpu.VMEM((CAP,), jnp.int32),
            pltpu.SemaphoreType.DMA,
            pltpu.SemaphoreType.DMA,
        ],
    )
    def k1(c_hbm, tbl_hbm, vals_hbm, jl_hbm, cch_v, oj_v, oc_v, slab_v,
           sj_v, sc_v, row_v, jlf_v, sem_slab, sem_out):
        wid = lax.axis_index("s") * NC + lax.axis_index("c")
        rbase = wid * RANGE
        span = jnp.maximum(jnp.minimum(V - rbase, RANGE), 0)
        ngrp = (span + GRP - 1) // GRP
        lanes = lax.iota(jnp.int32, 16)
        vbase = wid * CAP

        def grp_base(g):
            return pl.multiple_of(rbase + g * GRP, GRP)

        def sub_ok(g, d):
            return grp_base(g) + d * TILE <= LAST_TILE_BASE

        def slab_dma(g, d):
            return pltpu.make_async_copy(
                tbl_hbm.at[:, pl.ds(
                    pl.multiple_of(grp_base(g) + d * TILE, TILE), TILE)],
                slab_v.at[g & 1, :, pl.ds(d * TILE, TILE)],
                sem_slab,
            )

        def start_group(g):
            for d in range(4):
                @pl.when(sub_ok(g, d))
                def _():
                    slab_dma(g, d).start()

        def wait_group(g):
            for d in range(4):
                @pl.when(sub_ok(g, d))
                def _():
                    slab_dma(g, d).wait()

        def row_wait():
            pltpu.make_async_copy(
                row_v.at[pl.ds(0, 16 * D)],
                vals_hbm.at[pl.ds(0, 16 * D)], sem_out).wait()

        def emit(cnt, done_p, par, base):
            sj_v[pl.ds(cnt, 16)] = jnp.full((16,), B, jnp.int32)
            sc_v[pl.ds(cnt, 16)] = jnp.full((16,), base, jnp.int32)
            nblk16 = (cnt + 15) // 16

            def eblock(b, ecarry):
                par2 = b & 1

                @pl.when(b >= 2)
                def _():
                    row_wait()

                c16 = sc_v[pl.ds(b * 16, 16)]
                col16 = c16 - base

                for d in range(D):
                    vals = plsc.load_gather(
                        slab_v.at[par],
                        [jnp.full((16,), d, jnp.int32), col16])
                    plsc.store_scatter(
                        row_v, [par2 * (16 * D) + lanes * D + d], vals)
                dst0 = (vbase + done_p + b * 16) * D
                pltpu.async_copy(
                    row_v.at[pl.ds(par2 * (16 * D), 16 * D)],
                    vals_hbm.at[pl.ds(dst0, 16 * D)],
                    sem_out)
                jlf_v[pl.ds(done_p + b * 16, 16)] = sj_v[pl.ds(b * 16, 16)]
                return ecarry

            lax.fori_loop(0, nblk16, eblock, 0)
            ndrain = jnp.minimum(nblk16, 2)

            def drain_body(_i, _c2):
                row_wait()
                return _c2

            lax.fori_loop(0, ndrain, drain_body, 0)
            return done_p + cnt

        def sweep(total, done_in):
            for t in range(4):
                oc_v[pl.ds(total + t * 16, 16)] = jnp.full(
                    (16,), SENT, jnp.int32)
            nblk = (total + 15) // 16
            nblk64 = (total + 63) // 64

            @pl.when(ngrp > 0)
            def _():
                start_group(0)

            def group_body(g, done_g):
                wait_group(g)

                @pl.when(g + 1 < ngrp)
                def _():
                    start_group(g + 1)

                par = g & 1
                base = rbase + g * GRP
                gid = wid * NGRP + g

                def fast_blk(bb, c2):
                    cnt2, seen2 = c2
                    for u in range(4):
                        oc16 = oc_v[pl.ds(bb * 64 + u * 16, 16)]
                        m = (oc16 >> 9) == gid
                        n = jnp.sum(m.astype(jnp.int32))
                        ok = cnt2 < LCAP - 15
                        mm = m & ok
                        plsc.store_compressed(sc_v.at[pl.ds(cnt2, 16)],
                                              oc16, mask=mm)
                        oj16 = oj_v[pl.ds(bb * 64 + u * 16, 16)]
                        plsc.store_compressed(sj_v.at[pl.ds(cnt2, 16)],
                                              oj16, mask=mm)
                        cnt2 = cnt2 + jnp.where(ok, n, 0)
                        seen2 = seen2 + n
                    return cnt2, seen2

                cnt, gtotal = lax.fori_loop(
                    0, nblk64, fast_blk, (jnp.int32(0), jnp.int32(0)))
                done_g = emit(cnt, done_g, par, base)

                def ov_cond(st):
                    return st[0] < gtotal

                def ov_body(st):
                    kept, done2 = st

                    def rank_blk(bb, c2):
                        cnt2, seen2 = c2
                        oc16 = oc_v[pl.ds(bb * 16, 16)]
                        m = (oc16 >> 9) == gid
                        mi = m.astype(jnp.int32)
                        incl = plsc.cumsum(mi)
                        rank = seen2 + incl - mi
                        mm = m & (rank >= kept) & (rank < kept + LCAP)
                        n = jnp.sum(mm.astype(jnp.int32))
                        plsc.store_compressed(sc_v.at[pl.ds(cnt2, 16)],
                                              oc16, mask=mm)
                        oj16 = oj_v[pl.ds(bb * 16, 16)]
                        plsc.store_compressed(sj_v.at[pl.ds(cnt2, 16)],
                                              oj16, mask=mm)
                        return cnt2 + n, seen2 + jnp.sum(mi)

                    cnt2, _s = lax.fori_loop(0, nblk, rank_blk,
                                             (jnp.int32(0), jnp.int32(0)))
                    done3 = emit(cnt2, done2, par, base)
                    return kept + cnt2, done3

                kept_done = lax.while_loop(ov_cond, ov_body, (cnt, done_g))
                return kept_done[1]

            return lax.fori_loop(0, ngrp, group_body, done_in)

        def fast_chunk(ci, carry):
            off, seen = carry
            pltpu.sync_copy(c_hbm.at[pl.ds(ci * SCAN_CH, SCAN_CH)], cch_v)

            def vec_body(kk, c2):
                off2, seen2 = c2
                c16 = cch_v[pl.ds(kk * 16, 16)]
                m = (c16 >> RSHIFT) == wid
                n = jnp.sum(m.astype(jnp.int32))
                ok = off2 < OCAP - 15
                mm = m & ok
                plsc.store_compressed(oc_v.at[pl.ds(off2, 16)], c16,
                                      mask=mm)
                jv = lanes + (ci * SCAN_CH + kk * 16)
                plsc.store_compressed(oj_v.at[pl.ds(off2, 16)], jv,
                                      mask=mm)
                return off2 + jnp.where(ok, n, 0), seen2 + n

            return lax.fori_loop(0, SCAN_CH // 16, vec_body, (off, seen),
                                 unroll=4)

        off0, full_total = lax.fori_loop(
            0, NCHUNK, fast_chunk, (jnp.int32(0), jnp.int32(0)))
        done = sweep(off0, jnp.int32(0))

        def go_cond(st):
            return st[0] < full_total

        def go_body(st):
            kept, done2 = st

            def rank_chunk(ci, carry):
                off, seen = carry
                pltpu.sync_copy(c_hbm.at[pl.ds(ci * SCAN_CH, SCAN_CH)],
                                cch_v)

                def vec_body(kk, c2):
                    off2, seen2 = c2
                    c16 = cch_v[pl.ds(kk * 16, 16)]
                    m = (c16 >> RSHIFT) == wid
                    mi = m.astype(jnp.int32)
                    incl = plsc.cumsum(mi)
                    rank = seen2 + incl - mi
                    mm = m & (rank >= kept) & (rank < kept + OCAP)
                    n = jnp.sum(mm.astype(jnp.int32))
                    plsc.store_compressed(oc_v.at[pl.ds(off2, 16)], c16,
                                          mask=mm)
                    jv = lanes + (ci * SCAN_CH + kk * 16)
                    plsc.store_compressed(oj_v.at[pl.ds(off2, 16)], jv,
                                          mask=mm)
                    return off2 + n, seen2 + jnp.sum(mi)

                return lax.fori_loop(0, SCAN_CH // 16, vec_body,
                                     (off, seen))

            offn, _s = lax.fori_loop(0, NCHUNK, rank_chunk,
                                     (jnp.int32(0), jnp.int32(0)))
            done3 = sweep(offn, done2)
            return kept + offn, done3

        kept_done = lax.while_loop(go_cond, go_body, (off0, done))
        done = kept_done[1]

        for t in range(10):
            jlf_v[pl.ds(done + t * 16, 16)] = jnp.full((16,), B, jnp.int32)
        pltpu.sync_copy(jlf_v, jl_hbm.at[pl.ds(vbase, CAP)])

    return k1(c, table_t)


def _sc_scatter(c, vals2, jl):
    mesh = plsc.VectorSubcoreMesh(core_axis_name="c", subcore_axis_name="s")

    @functools.partial(
        pl.kernel,
        mesh=mesh,
        out_type=jax.ShapeDtypeStruct((B + 16, D), jnp.float32),
        compiler_params=pltpu.CompilerParams(
            use_tc_tiling_on_sc=False, needs_layout_passes=False),
        scratch_types=[
            pltpu.VMEM((2, TILE), jnp.int32),
            pltpu.VMEM((2, TILE, D), jnp.float32),
            pltpu.SemaphoreType.DMA,
            pltpu.SemaphoreType.DMA,
        ],
    )
    def k2(c_hbm, vals_hbm, jl_hbm, out_hbm, jl_v, row_v, sem_in, sem_sc):
        wid = lax.axis_index("s") * NC + lax.axis_index("c")
        vbase = wid * CAP
        nchmax = CAP // TILE

        def in_dma(ch):
            par = ch & 1
            a = pltpu.make_async_copy(
                jl_hbm.at[pl.ds(vbase + ch * TILE, TILE)],
                jl_v.at[par], sem_in)
            b = pltpu.make_async_copy(
                vals_hbm.at[pl.ds(vbase + ch * TILE, TILE), :],
                row_v.at[par], sem_in)
            return a, b

        def start_in(ch):
            a, b = in_dma(ch)
            a.start()
            b.start()

        def wait_in(ch):
            a, b = in_dma(ch)
            a.wait()
            b.wait()

        def scat_wait():
            pltpu.make_async_copy(
                row_v.at[0], out_hbm.at[jl_v.at[0]], sem_sc).wait()

        start_in(0)

        def w_cond(st):
            return st[1]

        def w_body(st):
            ch, _go, outst = st
            par = ch & 1
            wait_in(ch)
            j16 = jl_v[par, pl.ds(0, 16)]
            live = jnp.sum((j16 < B).astype(jnp.int32)) > 0

            @pl.when(outst >= 1)
            def _():
                scat_wait()

            outst = jnp.maximum(outst - 1, 0)

            @pl.when(live & (ch + 1 < nchmax))
            def _():
                start_in(ch + 1)

            @pl.when(live)
            def _():
                pltpu.async_copy(
                    row_v.at[par], out_hbm.at[jl_v.at[par]], sem_sc)

            return (ch + 1, live & (ch + 1 < nchmax),
                    outst + live.astype(jnp.int32))

        _ch, _go, outst_f = lax.while_loop(
            w_cond, w_body, (jnp.int32(0), jnp.bool_(True), jnp.int32(0)))
        ndrain = outst_f

        def drain_body(_i, _c):
            scat_wait()
            return _c

        lax.fori_loop(0, ndrain, drain_body, 0)

    return k2(c, vals2, jl)


def kernel(c, table):
    vals_flat, jl = _sc_collect(c, table.T)
    out = _sc_scatter(c, vals_flat.reshape(NW * CAP, D), jl)
    return out[:B]


# speedup vs baseline: 548.1834x; 1.0101x over previous
import functools

import jax
import jax.numpy as jnp
from jax import lax
from jax.experimental import pallas as pl
from jax.experimental.pallas import tpu as pltpu
from jax.experimental.pallas import tpu_sc as plsc

B = 16384
D = 64
V = 1000000
NC = 2
NS = 16
NW = NC * NS
RSHIFT = 15
RANGE = 1 << RSHIFT
GRP = 512
NGRP = RANGE // GRP
TILE = 128
SCAN_CH = 2048
NCHUNK = B // SCAN_CH
OCAP = 8192
LCAP = 2048
CAP = 16640
SENT = 0x7FFF0000
LAST_TILE_BASE = (V // TILE) * TILE


def _sc_collect(c, table_t):
    mesh = plsc.VectorSubcoreMesh(core_axis_name="c", subcore_axis_name="s")

    @functools.partial(
        pl.kernel,
        mesh=mesh,
        out_type=(
            jax.ShapeDtypeStruct((NW * CAP * D,), jnp.float32),
            jax.ShapeDtypeStruct((NW * CAP,), jnp.int32),
        ),
        compiler_params=pltpu.CompilerParams(
            use_tc_tiling_on_sc=True, needs_layout_passes=False),
        scratch_types=[
            pltpu.VMEM((SCAN_CH,), jnp.int32),
            pltpu.VMEM((OCAP + 96,), jnp.int32),
            pltpu.VMEM((OCAP + 96,), jnp.int32),
            pltpu.VMEM((2, D, GRP), jnp.float32),
            pltpu.VMEM((LCAP + 32,), jnp.int32),
            pltpu.VMEM((LCAP + 32,), jnp.int32),
            pltpu.VMEM((2 * 16 * D,), jnp.float32),
            pltpu.VMEM((CAP,), jnp.int32),
            pltpu.SemaphoreType.DMA,
            pltpu.SemaphoreType.DMA,
        ],
    )
    def k1(c_hbm, tbl_hbm, vals_hbm, jl_hbm, cch_v, oj_v, oc_v, slab_v,
           sj_v, sc_v, row_v, jlf_v, sem_slab, sem_out):
        wid = lax.axis_index("s") * NC + lax.axis_index("c")
        rbase = wid * RANGE
        span = jnp.maximum(jnp.minimum(V - rbase, RANGE), 0)
        ngrp = (span + GRP - 1) // GRP
        lanes = lax.iota(jnp.int32, 16)
        vbase = wid * CAP

        def grp_base(g):
            return pl.multiple_of(rbase + g * GRP, GRP)

        def sub_ok(g, d):
            return grp_base(g) + d * TILE <= LAST_TILE_BASE

        def slab_dma(g, d):
            return pltpu.make_async_copy(
                tbl_hbm.at[:, pl.ds(
                    pl.multiple_of(grp_base(g) + d * TILE, TILE), TILE)],
                slab_v.at[g & 1, :, pl.ds(d * TILE, TILE)],
                sem_slab,
            )

        def start_group(g):
            for d in range(4):
                @pl.when(sub_ok(g, d))
                def _():
                    slab_dma(g, d).start()

        def wait_group(g):
            for d in range(4):
                @pl.when(sub_ok(g, d))
                def _():
                    slab_dma(g, d).wait()

        def row_wait():
            pltpu.make_async_copy(
                row_v.at[pl.ds(0, 16 * D)],
                vals_hbm.at[pl.ds(0, 16 * D)], sem_out).wait()

        def emit(cnt, done_p, par, base):
            sj_v[pl.ds(cnt, 16)] = jnp.full((16,), B, jnp.int32)
            sc_v[pl.ds(cnt, 16)] = jnp.full((16,), base, jnp.int32)
            nblk16 = (cnt + 15) // 16

            def eblock(b, ecarry):
                par2 = b & 1

                @pl.when(b >= 2)
                def _():
                    row_wait()

                c16 = sc_v[pl.ds(b * 16, 16)]
                col16 = c16 - base

                for d in range(D):
                    vals = plsc.load_gather(
                        slab_v.at[par],
                        [jnp.full((16,), d, jnp.int32), col16])
                    plsc.store_scatter(
                        row_v, [par2 * (16 * D) + lanes * D + d], vals)
                dst0 = (vbase + done_p + b * 16) * D
                pltpu.async_copy(
                    row_v.at[pl.ds(par2 * (16 * D), 16 * D)],
                    vals_hbm.at[pl.ds(dst0, 16 * D)],
                    sem_out)
                jlf_v[pl.ds(done_p + b * 16, 16)] = sj_v[pl.ds(b * 16, 16)]
                return ecarry

            lax.fori_loop(0, nblk16, eblock, 0)
            ndrain = jnp.minimum(nblk16, 2)

            def drain_body(_i, _c2):
                row_wait()
                return _c2

            lax.fori_loop(0, ndrain, drain_body, 0)
            return done_p + cnt

        def sweep(total, done_in):
            for t in range(4):
                oc_v[pl.ds(total + t * 16, 16)] = jnp.full(
                    (16,), SENT, jnp.int32)
            nblk = (total + 15) // 16
            nblk64 = (total + 63) // 64

            @pl.when(ngrp > 0)
            def _():
                start_group(0)

            def group_body(g, done_g):
                wait_group(g)

                @pl.when(g + 1 < ngrp)
                def _():
                    start_group(g + 1)

                par = g & 1
                base = rbase + g * GRP
                gid = wid * NGRP + g

                def fast_blk(bb, c2):
                    cnt2, seen2 = c2
                    for u in range(4):
                        oc16 = oc_v[pl.ds(bb * 64 + u * 16, 16)]
                        m = (oc16 >> 9) == gid
                        n = plsc.all_reduce_population_count(m)[0]
                        ok = cnt2 < LCAP - 15
                        mm = m & ok
                        plsc.store_compressed(sc_v.at[pl.ds(cnt2, 16)],
                                              oc16, mask=mm)
                        oj16 = oj_v[pl.ds(bb * 64 + u * 16, 16)]
                        plsc.store_compressed(sj_v.at[pl.ds(cnt2, 16)],
                                              oj16, mask=mm)
                        cnt2 = cnt2 + jnp.where(ok, n, 0)
                        seen2 = seen2 + n
                    return cnt2, seen2

                cnt, gtotal = lax.fori_loop(
                    0, nblk64, fast_blk, (jnp.int32(0), jnp.int32(0)))
                done_g = emit(cnt, done_g, par, base)

                def ov_cond(st):
                    return st[0] < gtotal

                def ov_body(st):
                    kept, done2 = st

                    def rank_blk(bb, c2):
                        cnt2, seen2 = c2
                        oc16 = oc_v[pl.ds(bb * 16, 16)]
                        m = (oc16 >> 9) == gid
                        mi = m.astype(jnp.int32)
                        incl = plsc.cumsum(mi)
                        rank = seen2 + incl - mi
                        mm = m & (rank >= kept) & (rank < kept + LCAP)
                        n = jnp.sum(mm.astype(jnp.int32))
                        plsc.store_compressed(sc_v.at[pl.ds(cnt2, 16)],
                                              oc16, mask=mm)
                        oj16 = oj_v[pl.ds(bb * 16, 16)]
                        plsc.store_compressed(sj_v.at[pl.ds(cnt2, 16)],
                                              oj16, mask=mm)
                        return cnt2 + n, seen2 + jnp.sum(mi)

                    cnt2, _s = lax.fori_loop(0, nblk, rank_blk,
                                             (jnp.int32(0), jnp.int32(0)))
                    done3 = emit(cnt2, done2, par, base)
                    return kept + cnt2, done3

                kept_done = lax.while_loop(ov_cond, ov_body, (cnt, done_g))
                return kept_done[1]

            return lax.fori_loop(0, ngrp, group_body, done_in)

        def fast_chunk(ci, carry):
            off, seen = carry
            pltpu.sync_copy(c_hbm.at[pl.ds(ci * SCAN_CH, SCAN_CH)], cch_v)

            def vec_body(kk, c2):
                off2, seen2 = c2
                c16 = cch_v[pl.ds(kk * 16, 16)]
                m = (c16 >> RSHIFT) == wid
                n = plsc.all_reduce_population_count(m)[0]
                ok = off2 < OCAP - 15
                mm = m & ok
                plsc.store_compressed(oc_v.at[pl.ds(off2, 16)], c16,
                                      mask=mm)
                jv = lanes + (ci * SCAN_CH + kk * 16)
                plsc.store_compressed(oj_v.at[pl.ds(off2, 16)], jv,
                                      mask=mm)
                return off2 + jnp.where(ok, n, 0), seen2 + n

            return lax.fori_loop(0, SCAN_CH // 16, vec_body, (off, seen),
                                 unroll=4)

        off0, full_total = lax.fori_loop(
            0, NCHUNK, fast_chunk, (jnp.int32(0), jnp.int32(0)))
        done = sweep(off0, jnp.int32(0))

        def go_cond(st):
            return st[0] < full_total

        def go_body(st):
            kept, done2 = st

            def rank_chunk(ci, carry):
                off, seen = carry
                pltpu.sync_copy(c_hbm.at[pl.ds(ci * SCAN_CH, SCAN_CH)],
                                cch_v)

                def vec_body(kk, c2):
                    off2, seen2 = c2
                    c16 = cch_v[pl.ds(kk * 16, 16)]
                    m = (c16 >> RSHIFT) == wid
                    mi = m.astype(jnp.int32)
                    incl = plsc.cumsum(mi)
                    rank = seen2 + incl - mi
                    mm = m & (rank >= kept) & (rank < kept + OCAP)
                    n = jnp.sum(mm.astype(jnp.int32))
                    plsc.store_compressed(oc_v.at[pl.ds(off2, 16)], c16,
                                          mask=mm)
                    jv = lanes + (ci * SCAN_CH + kk * 16)
                    plsc.store_compressed(oj_v.at[pl.ds(off2, 16)], jv,
                                          mask=mm)
                    return off2 + n, seen2 + jnp.sum(mi)

                return lax.fori_loop(0, SCAN_CH // 16, vec_body,
                                     (off, seen))

            offn, _s = lax.fori_loop(0, NCHUNK, rank_chunk,
                                     (jnp.int32(0), jnp.int32(0)))
            done3 = sweep(offn, done2)
            return kept + offn, done3

        kept_done = lax.while_loop(go_cond, go_body, (off0, done))
        done = kept_done[1]

        for t in range(10):
            jlf_v[pl.ds(done + t * 16, 16)] = jnp.full((16,), B, jnp.int32)
        pltpu.sync_copy(jlf_v, jl_hbm.at[pl.ds(vbase, CAP)])

    return k1(c, table_t)


def _sc_scatter(c, vals2, jl):
    mesh = plsc.VectorSubcoreMesh(core_axis_name="c", subcore_axis_name="s")

    @functools.partial(
        pl.kernel,
        mesh=mesh,
        out_type=jax.ShapeDtypeStruct((B + 16, D), jnp.float32),
        compiler_params=pltpu.CompilerParams(
            use_tc_tiling_on_sc=False, needs_layout_passes=False),
        scratch_types=[
            pltpu.VMEM((2, TILE), jnp.int32),
            pltpu.VMEM((2, TILE, D), jnp.float32),
            pltpu.SemaphoreType.DMA,
            pltpu.SemaphoreType.DMA,
        ],
    )
    def k2(c_hbm, vals_hbm, jl_hbm, out_hbm, jl_v, row_v, sem_in, sem_sc):
        wid = lax.axis_index("s") * NC + lax.axis_index("c")
        vbase = wid * CAP
        nchmax = CAP // TILE

        def in_dma(ch):
            par = ch & 1
            a = pltpu.make_async_copy(
                jl_hbm.at[pl.ds(vbase + ch * TILE, TILE)],
                jl_v.at[par], sem_in)
            b = pltpu.make_async_copy(
                vals_hbm.at[pl.ds(vbase + ch * TILE, TILE), :],
                row_v.at[par], sem_in)
            return a, b

        def start_in(ch):
            a, b = in_dma(ch)
            a.start()
            b.start()

        def wait_in(ch):
            a, b = in_dma(ch)
            a.wait()
            b.wait()

        def scat_wait():
            pltpu.make_async_copy(
                row_v.at[0], out_hbm.at[jl_v.at[0]], sem_sc).wait()

        start_in(0)

        def w_cond(st):
            return st[1]

        def w_body(st):
            ch, _go, outst = st
            par = ch & 1
            wait_in(ch)
            j16 = jl_v[par, pl.ds(0, 16)]
            live = plsc.all_reduce_population_count(j16 < B)[0] > 0

            @pl.when(outst >= 1)
            def _():
                scat_wait()

            outst = jnp.maximum(outst - 1, 0)

            @pl.when(live & (ch + 1 < nchmax))
            def _():
                start_in(ch + 1)

            @pl.when(live)
            def _():
                pltpu.async_copy(
                    row_v.at[par], out_hbm.at[jl_v.at[par]], sem_sc)

            return (ch + 1, live & (ch + 1 < nchmax),
                    outst + live.astype(jnp.int32))

        _ch, _go, outst_f = lax.while_loop(
            w_cond, w_body, (jnp.int32(0), jnp.bool_(True), jnp.int32(0)))
        ndrain = outst_f

        def drain_body(_i, _c):
            scat_wait()
            return _c

        lax.fori_loop(0, ndrain, drain_body, 0)

    return k2(c, vals2, jl)


def kernel(c, table):
    vals_flat, jl = _sc_collect(c, table.T)
    out = _sc_scatter(c, vals_flat.reshape(NW * CAP, D), jl)
    return out[:B]
